# baseline (speedup 1.0000x reference)
import functools

import jax
import jax.numpy as jnp
from jax import lax
from jax.experimental import pallas as pl
from jax.experimental.pallas import tpu as pltpu
from jax.experimental.pallas import tpu_sc as plsc

_NC, _NS = 2, 16
_NW = _NC * _NS
_G = 128


def _sc_gather(emb_flat, w_flat, idx_grp):
    T, D = emb_flat.shape
    ngrp = idx_grp.shape[0]
    n_rows = ngrp * _G
    assert ngrp % _NW == 0
    ngrp_w = ngrp // _NW
    cg = 8
    assert ngrp_w % cg == 0
    nch = ngrp_w // cg
    ch = cg * _G

    mesh = plsc.VectorSubcoreMesh(
        core_axis_name="c", subcore_axis_name="s",
        num_cores=_NC, num_subcores=_NS)

    @functools.partial(
        pl.kernel, mesh=mesh,
        out_type=(jax.ShapeDtypeStruct((n_rows, D), jnp.float32),
                  jax.ShapeDtypeStruct((n_rows,), jnp.float32)),
        scratch_types=[
            pltpu.VMEM((cg, _G), jnp.int32),
            pltpu.VMEM((ch, D), jnp.float32),
            pltpu.VMEM((ch,), jnp.float32),
            pltpu.SemaphoreType.DMA,
            pltpu.SemaphoreType.DMA,
        ],
        compiler_params=pltpu.CompilerParams(use_tc_tiling_on_sc=False),
    )
    def k(emb_hbm, w_hbm, idx_hbm, eout_hbm, wout_hbm, idx_v, emb_v, w_v, esem, wsem):
        wid = lax.axis_index("s") * _NC + lax.axis_index("c")
        gbase = wid * ngrp_w

        @pl.loop(0, nch)
        def _chunk(ci):
            g0 = gbase + ci * cg
            pltpu.sync_copy(idx_hbm.at[pl.ds(g0, cg)], idx_v)
            descs = []
            for g in range(cg):
                descs.append(pltpu.async_copy(
                    emb_hbm.at[idx_v.at[g]], emb_v.at[pl.ds(g * _G, _G)], esem))
                descs.append(pltpu.async_copy(
                    w_hbm.at[idx_v.at[g]], w_v.at[pl.ds(g * _G, _G)], wsem))
            for d_ in descs:
                d_.wait()
            r0 = g0 * _G
            pltpu.sync_copy(emb_v, eout_hbm.at[pl.ds(r0, ch)])
            pltpu.sync_copy(w_v, wout_hbm.at[pl.ds(r0, ch)])

    return k(emb_flat, w_flat, idx_grp)


_TB = 256


def _tc1(emb, dense, wg, dlW, ddW, ddb, W1, b1, c0):
    Bsz, FD = emb.shape
    Dn = dense.shape[1]
    F = wg.shape[1]
    D = FD // F
    H1 = W1.shape[0]
    nt = Bsz // _TB

    def body(emb_ref, dense_ref, wg_ref, dlW_ref, ddW_ref, ddb_ref, W1_ref,
             b1_ref, c0_ref, z1_ref, s1_ref, ss1_ref, lf_ref):
        i = pl.program_id(0)
        emb_t = emb_ref[...]
        dense_t = dense_ref[...]
        r = lax.broadcasted_iota(jnp.int32, (FD, D), 0)
        dcol = lax.broadcasted_iota(jnp.int32, (FD, D), 1)
        sel = (r % D == dcol).astype(jnp.float32)
        s = lax.dot_general(emb_t, sel, (((1,), (0,)), ((), ())),
                            preferred_element_type=jnp.float32)
        sq_sum = jnp.sum(emb_t * emb_t, axis=1, keepdims=True)
        fm = 0.5 * (jnp.sum(s * s, axis=1, keepdims=True) - sq_sum)
        lin = (c0_ref[...] +
               jnp.sum(wg_ref[...], axis=1, keepdims=True) +
               lax.dot_general(dense_t, dlW_ref[...], (((1,), (1,)), ((), ())),
                               preferred_element_type=jnp.float32))
        lf_ref[...] = lin + fm
        dense_emb = lax.dot_general(dense_t, ddW_ref[...], (((1,), (1,)), ((), ())),
                                    preferred_element_type=jnp.float32) + ddb_ref[...]
        dnn_in = jnp.concatenate([emb_t, dense_emb], axis=1)
        z1 = lax.dot_general(dnn_in, W1_ref[...], (((1,), (1,)), ((), ())),
                             preferred_element_type=jnp.float32) + b1_ref[...]
        z1_ref[...] = z1

        @pl.when(i == 0)
        def _():
            s1_ref[...] = jnp.zeros_like(s1_ref)
            ss1_ref[...] = jnp.zeros_like(ss1_ref)

        s1_ref[...] += jnp.sum(z1, axis=0, keepdims=True)
        ss1_ref[...] += jnp.sum(z1 * z1, axis=0, keepdims=True)

    full = lambda shp: pl.BlockSpec(shp, lambda i: (0, 0))
    return pl.pallas_call(
        body,
        grid=(nt,),
        in_specs=[
            pl.BlockSpec((_TB, FD), lambda i: (i, 0)),
            pl.BlockSpec((_TB, Dn), lambda i: (i, 0)),
            pl.BlockSpec((_TB, F), lambda i: (i, 0)),
            full((1, Dn)), full((D, Dn)), full((1, D)),
            full((H1, FD + D)), full((1, H1)), full((1, 1)),
        ],
        out_specs=[
            pl.BlockSpec((_TB, H1), lambda i: (i, 0)),
            full((1, H1)), full((1, H1)),
            pl.BlockSpec((_TB, 1), lambda i: (i, 0)),
        ],
        out_shape=[
            jax.ShapeDtypeStruct((Bsz, H1), jnp.float32),
            jax.ShapeDtypeStruct((1, H1), jnp.float32),
            jax.ShapeDtypeStruct((1, H1), jnp.float32),
            jax.ShapeDtypeStruct((Bsz, 1), jnp.float32),
        ],
        compiler_params=pltpu.CompilerParams(dimension_semantics=("arbitrary",)),
    )(emb, dense, wg, dlW, ddW, ddb, W1, b1, c0)


def _bn_relu(z, s, ss, g, be, nB):
    mu = s * (1.0 / nB)
    var = ss * (1.0 / nB) - mu * mu
    inv = lax.rsqrt(var + 1e-5) * g
    return jnp.maximum(z * inv + (be - mu * inv), 0.0)


def _tc_mid(z, s, ss, g, be, W, b):
    Bsz, din = z.shape
    dout = W.shape[0]
    nt = Bsz // _TB

    def body(z_ref, s_ref, ss_ref, g_ref, be_ref, W_ref, b_ref,
             zo_ref, so_ref, sso_ref):
        i = pl.program_id(0)
        h = _bn_relu(z_ref[...], s_ref[...], ss_ref[...], g_ref[...],
                     be_ref[...], float(Bsz))
        zo = lax.dot_general(h, W_ref[...], (((1,), (1,)), ((), ())),
                             preferred_element_type=jnp.float32) + b_ref[...]
        zo_ref[...] = zo

        @pl.when(i == 0)
        def _():
            so_ref[...] = jnp.zeros_like(so_ref)
            sso_ref[...] = jnp.zeros_like(sso_ref)

        so_ref[...] += jnp.sum(zo, axis=0, keepdims=True)
        sso_ref[...] += jnp.sum(zo * zo, axis=0, keepdims=True)

    full = lambda shp: pl.BlockSpec(shp, lambda i: (0, 0))
    return pl.pallas_call(
        body,
        grid=(nt,),
        in_specs=[
            pl.BlockSpec((_TB, din), lambda i: (i, 0)),
            full((1, din)), full((1, din)), full((1, din)), full((1, din)),
            full((dout, din)), full((1, dout)),
        ],
        out_specs=[
            pl.BlockSpec((_TB, dout), lambda i: (i, 0)),
            full((1, dout)), full((1, dout)),
        ],
        out_shape=[
            jax.ShapeDtypeStruct((Bsz, dout), jnp.float32),
            jax.ShapeDtypeStruct((1, dout), jnp.float32),
            jax.ShapeDtypeStruct((1, dout), jnp.float32),
        ],
        compiler_params=pltpu.CompilerParams(dimension_semantics=("arbitrary",)),
    )(z, s, ss, g, be, W, b)


def _tc_final(z, s, ss, g, be, Wo, c1, lf):
    Bsz, din = z.shape
    nt = Bsz // _TB

    def body(z_ref, s_ref, ss_ref, g_ref, be_ref, Wo_ref, c1_ref, lf_ref, o_ref):
        h = _bn_relu(z_ref[...], s_ref[...], ss_ref[...], g_ref[...],
                     be_ref[...], float(Bsz))
        dnn = lax.dot_general(h, Wo_ref[...], (((1,), (1,)), ((), ())),
                              preferred_element_type=jnp.float32)
        logits = lf_ref[...] + dnn + c1_ref[...]
        o_ref[...] = 1.0 / (1.0 + jnp.exp(-logits))

    full = lambda shp: pl.BlockSpec(shp, lambda i: (0, 0))
    return pl.pallas_call(
        body,
        grid=(nt,),
        in_specs=[
            pl.BlockSpec((_TB, din), lambda i: (i, 0)),
            full((1, din)), full((1, din)), full((1, din)), full((1, din)),
            full((1, din)), full((1, 1)),
            pl.BlockSpec((_TB, 1), lambda i: (i, 0)),
        ],
        out_specs=pl.BlockSpec((_TB, 1), lambda i: (i, 0)),
        out_shape=jax.ShapeDtypeStruct((Bsz, 1), jnp.float32),
        compiler_params=pltpu.CompilerParams(dimension_semantics=("arbitrary",)),
    )(z, s, ss, g, be, Wo, c1, lf)


def kernel(sparse_features, dense_features, emb_tables, weight_tables,
           dense_lin_W, dense_lin_b, dense_dnn_W, dense_dnn_b,
           W1, b1, g1, be1, W2, b2, g2, be2, W3, b3, g3, be3, Wo, bo, bias):
    B, F = sparse_features.shape
    V, D = emb_tables.shape[1], emb_tables.shape[2]
    n_rows = B * F

    idx = (sparse_features.astype(jnp.int32) +
           (jnp.arange(F, dtype=jnp.int32) * V)[None, :])
    idx_grp = idx.reshape(n_rows // _G, _G)

    emb_rows, w_rows = _sc_gather(
        emb_tables.reshape(F * V, D), weight_tables.reshape(F * V), idx_grp)
    emb = emb_rows.reshape(B, F * D)
    wg = w_rows.reshape(B, F)

    c0 = (bias + dense_lin_b).reshape(1, 1)
    z1, s1, ss1, lf = _tc1(emb, dense_features, wg, dense_lin_W, dense_dnn_W,
                           dense_dnn_b.reshape(1, -1), W1, b1.reshape(1, -1), c0)
    z2, s2, ss2 = _tc_mid(z1, s1, ss1, g1.reshape(1, -1), be1.reshape(1, -1),
                          W2, b2.reshape(1, -1))
    z3, s3, ss3 = _tc_mid(z2, s2, ss2, g2.reshape(1, -1), be2.reshape(1, -1),
                          W3, b3.reshape(1, -1))
    out = _tc_final(z3, s3, ss3, g3.reshape(1, -1), be3.reshape(1, -1),
                    Wo, bo.reshape(1, 1), lf)
    return out.reshape(B)


# speedup vs baseline: 1.0647x; 1.0647x over previous
import functools

import jax
import jax.numpy as jnp
from jax import lax
from jax.experimental import pallas as pl
from jax.experimental.pallas import tpu as pltpu
from jax.experimental.pallas import tpu_sc as plsc

_NC, _NS = 2, 16
_NW = _NC * _NS
_G = 128
_L = 16


def _sc_gather(emb_flat, w_flat, idx_grp, F):
    T, D = emb_flat.shape
    ngrp = idx_grp.shape[0]
    n_rows = ngrp * _G
    B = n_rows // F
    assert ngrp % _NW == 0 and B % _NW == 0
    ngrp_w = ngrp // _NW
    bpw = B // _NW
    cg = 8
    assert ngrp_w % cg == 0
    nch = ngrp_w // cg
    ch = cg * _G
    assert (ngrp_w * _G) % F == 0

    mesh = plsc.VectorSubcoreMesh(
        core_axis_name="c", subcore_axis_name="s",
        num_cores=_NC, num_subcores=_NS)

    @functools.partial(
        pl.kernel, mesh=mesh,
        out_type=(jax.ShapeDtypeStruct((n_rows, D), jnp.float32),
                  jax.ShapeDtypeStruct((B,), jnp.float32)),
        scratch_types=[
            pltpu.VMEM((cg, _G), jnp.int32),
            pltpu.VMEM((ch, D), jnp.float32),
            pltpu.VMEM((ngrp_w * _G,), jnp.float32),
            pltpu.VMEM((bpw,), jnp.float32),
            pltpu.SemaphoreType.DMA,
            pltpu.SemaphoreType.DMA,
        ],
        compiler_params=pltpu.CompilerParams(
            use_tc_tiling_on_sc=False, needs_layout_passes=False),
    )
    def k(emb_hbm, w_hbm, idx_hbm, eout_hbm, wsum_hbm,
          idx_v, emb_v, w_v, wsum_v, esem, wsem):
        wid = lax.axis_index("s") * _NC + lax.axis_index("c")
        gbase = wid * ngrp_w
        pbase = gbase * _G
        bbase = pbase // F

        @pl.loop(0, nch)
        def _chunk(ci):
            g0 = gbase + ci * cg
            pltpu.sync_copy(idx_hbm.at[pl.ds(g0, cg)], idx_v)
            descs = []
            for g in range(cg):
                descs.append(pltpu.async_copy(
                    emb_hbm.at[idx_v.at[g]], emb_v.at[pl.ds(g * _G, _G)], esem))
                descs.append(pltpu.async_copy(
                    w_hbm.at[idx_v.at[g]],
                    w_v.at[pl.ds(ci * ch + g * _G, _G)], wsem))
            for d_ in descs:
                d_.wait()
            r0 = g0 * _G
            pltpu.sync_copy(emb_v, eout_hbm.at[pl.ds(r0, ch)])

        lane_f = lax.broadcasted_iota(jnp.int32, (_L,), 0) * F
        for grp in range(bpw // _L):
            acc = jnp.zeros((_L,), jnp.float32)
            for f in range(F):
                acc = acc + plsc.load_gather(w_v, [grp * _L * F + f + lane_f])
            wsum_v[pl.ds(grp * _L, _L)] = acc

        pltpu.sync_copy(wsum_v, wsum_hbm.at[pl.ds(wid * bpw, bpw)])

    return k(emb_flat, w_flat, idx_grp)


_TB = 512


def _tc1(emb, dense, dlW, ddW, ddb, W1, b1, c0):
    Bsz, FD = emb.shape
    Dn = dense.shape[1]
    D = ddb.shape[1]
    H1 = W1.shape[0]
    nt = Bsz // _TB

    def body(emb_ref, dense_ref, dlW_ref, ddW_ref, ddb_ref, W1_ref,
             b1_ref, c0_ref, z1_ref, s1_ref, ss1_ref, lf_ref):
        i = pl.program_id(0)
        emb_t = emb_ref[...]
        dense_t = dense_ref[...]
        r = lax.broadcasted_iota(jnp.int32, (FD, D), 0)
        dcol = lax.broadcasted_iota(jnp.int32, (FD, D), 1)
        sel = (r % D == dcol).astype(jnp.float32)
        s = lax.dot_general(emb_t, sel, (((1,), (0,)), ((), ())),
                            preferred_element_type=jnp.float32)
        sq_sum = jnp.sum(emb_t * emb_t, axis=1, keepdims=True)
        fm = 0.5 * (jnp.sum(s * s, axis=1, keepdims=True) - sq_sum)
        lin = (c0_ref[...] +
               lax.dot_general(dense_t, dlW_ref[...], (((1,), (1,)), ((), ())),
                               preferred_element_type=jnp.float32))
        lf_ref[...] = lin + fm
        dense_emb = lax.dot_general(dense_t, ddW_ref[...], (((1,), (1,)), ((), ())),
                                    preferred_element_type=jnp.float32) + ddb_ref[...]
        dnn_in = jnp.concatenate([emb_t, dense_emb], axis=1)
        z1 = lax.dot_general(dnn_in, W1_ref[...], (((1,), (1,)), ((), ())),
                             preferred_element_type=jnp.float32) + b1_ref[...]
        z1_ref[...] = z1

        @pl.when(i == 0)
        def _():
            s1_ref[...] = jnp.zeros_like(s1_ref)
            ss1_ref[...] = jnp.zeros_like(ss1_ref)

        s1_ref[...] += jnp.sum(z1, axis=0, keepdims=True)
        ss1_ref[...] += jnp.sum(z1 * z1, axis=0, keepdims=True)

    full = lambda shp: pl.BlockSpec(shp, lambda i: (0, 0))
    return pl.pallas_call(
        body,
        grid=(nt,),
        in_specs=[
            pl.BlockSpec((_TB, FD), lambda i: (i, 0)),
            pl.BlockSpec((_TB, Dn), lambda i: (i, 0)),
            full((1, Dn)), full((D, Dn)), full((1, D)),
            full((H1, FD + D)), full((1, H1)), full((1, 1)),
        ],
        out_specs=[
            pl.BlockSpec((_TB, H1), lambda i: (i, 0)),
            full((1, H1)), full((1, H1)),
            pl.BlockSpec((_TB, 1), lambda i: (i, 0)),
        ],
        out_shape=[
            jax.ShapeDtypeStruct((Bsz, H1), jnp.float32),
            jax.ShapeDtypeStruct((1, H1), jnp.float32),
            jax.ShapeDtypeStruct((1, H1), jnp.float32),
            jax.ShapeDtypeStruct((Bsz, 1), jnp.float32),
        ],
        compiler_params=pltpu.CompilerParams(dimension_semantics=("arbitrary",)),
    )(emb, dense, dlW, ddW, ddb, W1, b1, c0)


def _bn_relu(z, s, ss, g, be, nB):
    mu = s * (1.0 / nB)
    var = ss * (1.0 / nB) - mu * mu
    inv = lax.rsqrt(var + 1e-5) * g
    return jnp.maximum(z * inv + (be - mu * inv), 0.0)


def _tc_mid(z, s, ss, g, be, W, b):
    Bsz, din = z.shape
    dout = W.shape[0]
    nt = Bsz // _TB

    def body(z_ref, s_ref, ss_ref, g_ref, be_ref, W_ref, b_ref,
             zo_ref, so_ref, sso_ref):
        i = pl.program_id(0)
        h = _bn_relu(z_ref[...], s_ref[...], ss_ref[...], g_ref[...],
                     be_ref[...], float(Bsz))
        zo = lax.dot_general(h, W_ref[...], (((1,), (1,)), ((), ())),
                             preferred_element_type=jnp.float32) + b_ref[...]
        zo_ref[...] = zo

        @pl.when(i == 0)
        def _():
            so_ref[...] = jnp.zeros_like(so_ref)
            sso_ref[...] = jnp.zeros_like(sso_ref)

        so_ref[...] += jnp.sum(zo, axis=0, keepdims=True)
        sso_ref[...] += jnp.sum(zo * zo, axis=0, keepdims=True)

    full = lambda shp: pl.BlockSpec(shp, lambda i: (0, 0))
    return pl.pallas_call(
        body,
        grid=(nt,),
        in_specs=[
            pl.BlockSpec((_TB, din), lambda i: (i, 0)),
            full((1, din)), full((1, din)), full((1, din)), full((1, din)),
            full((dout, din)), full((1, dout)),
        ],
        out_specs=[
            pl.BlockSpec((_TB, dout), lambda i: (i, 0)),
            full((1, dout)), full((1, dout)),
        ],
        out_shape=[
            jax.ShapeDtypeStruct((Bsz, dout), jnp.float32),
            jax.ShapeDtypeStruct((1, dout), jnp.float32),
            jax.ShapeDtypeStruct((1, dout), jnp.float32),
        ],
        compiler_params=pltpu.CompilerParams(dimension_semantics=("arbitrary",)),
    )(z, s, ss, g, be, W, b)


def _tc_final(z, s, ss, g, be, Wo, c1, lf, wsum):
    Bsz, din = z.shape
    nt = Bsz // _TB

    def body(z_ref, s_ref, ss_ref, g_ref, be_ref, Wo_ref, c1_ref, lf_ref,
             wsum_ref, o_ref):
        h = _bn_relu(z_ref[...], s_ref[...], ss_ref[...], g_ref[...],
                     be_ref[...], float(Bsz))
        dnn = lax.dot_general(h, Wo_ref[...], (((1,), (1,)), ((), ())),
                              preferred_element_type=jnp.float32)
        logits = (lf_ref[...] + dnn + c1_ref[...] +
                  jnp.expand_dims(wsum_ref[...], 1))
        o_ref[...] = 1.0 / (1.0 + jnp.exp(-logits))

    full = lambda shp: pl.BlockSpec(shp, lambda i: (0, 0))
    return pl.pallas_call(
        body,
        grid=(nt,),
        in_specs=[
            pl.BlockSpec((_TB, din), lambda i: (i, 0)),
            full((1, din)), full((1, din)), full((1, din)), full((1, din)),
            full((1, din)), full((1, 1)),
            pl.BlockSpec((_TB, 1), lambda i: (i, 0)),
            pl.BlockSpec((_TB,), lambda i: (i,)),
        ],
        out_specs=pl.BlockSpec((_TB, 1), lambda i: (i, 0)),
        out_shape=jax.ShapeDtypeStruct((Bsz, 1), jnp.float32),
        compiler_params=pltpu.CompilerParams(dimension_semantics=("arbitrary",)),
    )(z, s, ss, g, be, Wo, c1, lf, wsum)


def kernel(sparse_features, dense_features, emb_tables, weight_tables,
           dense_lin_W, dense_lin_b, dense_dnn_W, dense_dnn_b,
           W1, b1, g1, be1, W2, b2, g2, be2, W3, b3, g3, be3, Wo, bo, bias):
    B, F = sparse_features.shape
    V, D = emb_tables.shape[1], emb_tables.shape[2]
    n_rows = B * F

    idx = (sparse_features.astype(jnp.int32) +
           (jnp.arange(F, dtype=jnp.int32) * V)[None, :])
    idx_grp = idx.reshape(n_rows // _G, _G)

    emb_rows, wsum = _sc_gather(
        emb_tables.reshape(F * V, D), weight_tables.reshape(F * V), idx_grp, F)
    emb = emb_rows.reshape(B, F * D)

    c0 = (bias + dense_lin_b).reshape(1, 1)
    z1, s1, ss1, lf = _tc1(emb, dense_features, dense_lin_W, dense_dnn_W,
                           dense_dnn_b.reshape(1, -1), W1, b1.reshape(1, -1), c0)
    z2, s2, ss2 = _tc_mid(z1, s1, ss1, g1.reshape(1, -1), be1.reshape(1, -1),
                          W2, b2.reshape(1, -1))
    z3, s3, ss3 = _tc_mid(z2, s2, ss2, g2.reshape(1, -1), be2.reshape(1, -1),
                          W3, b3.reshape(1, -1))
    out = _tc_final(z3, s3, ss3, g3.reshape(1, -1), be3.reshape(1, -1),
                    Wo, bo.reshape(1, 1), lf, wsum)
    return out.reshape(B)
